# baseline (speedup 1.0000x reference)
import functools

import jax
import jax.numpy as jnp
from jax import lax
from jax.experimental import pallas as pl
from jax.experimental.pallas import tpu as pltpu
from jax.experimental.pallas import tpu_sc as plsc

VOCAB = 1000000
D = 64
B = 4096
L = 200
BF = B * L

NC = 2
NS = 16
NW = NC * NS
PER_W = BF // NW

S = 512
NSTREAM = PER_W // S
G = 1
CH = G * S
NCHUNK = NSTREAM // G

_mesh = plsc.VectorSubcoreMesh(core_axis_name="c", subcore_axis_name="s")


@functools.partial(
    pl.kernel,
    mesh=_mesh,
    out_type=jax.ShapeDtypeStruct((BF, D), jnp.float32),
    scratch_types=[
        pltpu.VMEM((NSTREAM, S), jnp.int32),
        pltpu.VMEM((CH, D), jnp.float32),
        pltpu.VMEM((CH, D), jnp.float32),
        pltpu.SemaphoreType.DMA,
        pltpu.SemaphoreType.DMA,
        pltpu.SemaphoreType.DMA,
        pltpu.SemaphoreType.DMA,
    ],
    compiler_params=pltpu.CompilerParams(use_tc_tiling_on_sc=False),
)
def _embed_sc(x_hbm, table_hbm, out_hbm, idx_v, rows0, rows1,
              gsem0, gsem1, ssem0, ssem1):
    wid = lax.axis_index("s") * NC + lax.axis_index("c")
    base = wid * PER_W

    pltpu.sync_copy(x_hbm.at[wid], idx_v)

    rows = (rows0, rows1)
    gsem = (gsem0, gsem1)
    ssem = (ssem0, ssem1)

    def fire_gathers(cc, b):
        for j in range(G):
            pltpu.async_copy(
                table_hbm.at[idx_v.at[cc * G + j]],
                rows[b].at[pl.ds(j * S, S)],
                gsem[b],
            )

    def wait_gathers(cc, b):
        for j in range(G):
            pltpu.make_async_copy(
                table_hbm.at[idx_v.at[cc * G + j]],
                rows[b].at[pl.ds(j * S, S)],
                gsem[b],
            ).wait()

    def out_ref(cc):
        return out_hbm.at[pl.ds(base + cc * CH, CH)]

    fire_gathers(0, 0)

    def chunk_body(cc, b):
        @pl.when(cc >= 1)
        def _():
            pltpu.make_async_copy(rows[1 - b], out_ref(cc - 1),
                                  ssem[1 - b]).wait()

        @pl.when(cc + 1 < NCHUNK)
        def _():
            fire_gathers(cc + 1, 1 - b)

        wait_gathers(cc, b)
        pltpu.async_copy(rows[b], out_ref(cc), ssem[b])

    def outer(i, carry):
        cc = i * 2
        chunk_body(cc, 0)
        chunk_body(cc + 1, 1)
        return carry

    lax.fori_loop(0, NCHUNK // 2, outer, 0)

    pltpu.make_async_copy(rows[1], out_ref(NCHUNK - 1), ssem[1]).wait()


def kernel(x, table):
    xf = x.astype(jnp.int32).reshape(NW, NSTREAM, S)
    out = _embed_sc(xf, table)
    return out.reshape(B, L, D)


# speedup vs baseline: 1.2192x; 1.2192x over previous
import functools

import jax
import jax.numpy as jnp
from jax import lax
from jax.experimental import pallas as pl
from jax.experimental.pallas import tpu as pltpu
from jax.experimental.pallas import tpu_sc as plsc

VOCAB = 1000000
D = 64
DP = 128
B = 4096
L = 200
BF = B * L

NC = 2
NS = 16
NW = NC * NS
BPW = B // NW

NCHUNK = BPW

_mesh = plsc.VectorSubcoreMesh(core_axis_name="c", subcore_axis_name="s")


@functools.partial(
    pl.kernel,
    mesh=_mesh,
    out_type=jax.ShapeDtypeStruct((BF, DP), jnp.float32),
    scratch_types=[
        pltpu.VMEM((BPW, L), jnp.int32),
        pltpu.VMEM((L, DP), jnp.float32),
        pltpu.VMEM((L, DP), jnp.float32),
        pltpu.SemaphoreType.DMA,
        pltpu.SemaphoreType.DMA,
        pltpu.SemaphoreType.DMA,
        pltpu.SemaphoreType.DMA,
    ],
    compiler_params=pltpu.CompilerParams(use_tc_tiling_on_sc=False),
)
def _embed_sc(x_hbm, table_hbm, out_hbm, idx_v, rows0, rows1,
              gsem0, gsem1, ssem0, ssem1):
    wid = lax.axis_index("s") * NC + lax.axis_index("c")
    base = wid * BPW

    pltpu.sync_copy(x_hbm.at[pl.ds(base, BPW)], idx_v)

    rows = (rows0, rows1)
    gsem = (gsem0, gsem1)
    ssem = (ssem0, ssem1)

    def fire_gather(cc, b):
        pltpu.async_copy(table_hbm.at[idx_v.at[cc]], rows[b], gsem[b])

    def wait_gather(cc, b):
        pltpu.make_async_copy(table_hbm.at[idx_v.at[cc]], rows[b],
                              gsem[b]).wait()

    def out_ref(cc):
        return out_hbm.at[pl.ds((base + cc) * L, L)]

    fire_gather(0, 0)

    def chunk_body(cc, b):
        @pl.when(cc >= 1)
        def _():
            pltpu.make_async_copy(rows[1 - b], out_ref(cc - 1),
                                  ssem[1 - b]).wait()

        @pl.when(cc + 1 < NCHUNK)
        def _():
            fire_gather(cc + 1, 1 - b)

        wait_gather(cc, b)
        pltpu.async_copy(rows[b], out_ref(cc), ssem[b])

    def outer(i, carry):
        cc = i * 2
        chunk_body(cc, 0)
        chunk_body(cc + 1, 1)
        return carry

    lax.fori_loop(0, NCHUNK // 2, outer, 0)

    pltpu.make_async_copy(rows[1], out_ref(NCHUNK - 1), ssem[1]).wait()


def kernel(x, table):
    tp = jnp.pad(table, ((0, 0), (0, DP - D)))
    outp = _embed_sc(x.astype(jnp.int32), tp)
    return outp.reshape(B, L, DP)[:, :, :D]


# speedup vs baseline: 1.3089x; 1.0736x over previous
import functools

import jax
import jax.numpy as jnp
from jax import lax
from jax.experimental import pallas as pl
from jax.experimental.pallas import tpu as pltpu
from jax.experimental.pallas import tpu_sc as plsc

VOCAB = 1000000
D = 64
DP = 128
B = 4096
L = 200
BF = B * L

NC = 2
NS = 16
NW = NC * NS
BPW = B // NW

NCHUNK = BPW

_mesh = plsc.VectorSubcoreMesh(core_axis_name="c", subcore_axis_name="s")


@functools.partial(
    pl.kernel,
    mesh=_mesh,
    out_type=jax.ShapeDtypeStruct((BF, DP), jnp.float32),
    scratch_types=[
        pltpu.VMEM((BPW, L), jnp.int32),
        pltpu.VMEM((L, DP), jnp.float32),
        pltpu.VMEM((L, DP), jnp.float32),
        pltpu.SemaphoreType.DMA,
        pltpu.SemaphoreType.DMA,
        pltpu.SemaphoreType.DMA,
        pltpu.SemaphoreType.DMA,
    ],
    compiler_params=pltpu.CompilerParams(use_tc_tiling_on_sc=False),
)
def _embed_sc(x_hbm, table_hbm, out_hbm, idx_v, rows0, rows1,
              gsem0, gsem1, ssem0, ssem1):
    wid = lax.axis_index("s") * NC + lax.axis_index("c")
    base = wid * BPW

    pltpu.sync_copy(x_hbm.at[pl.ds(base, BPW)], idx_v)

    rows = (rows0, rows1)
    gsem = (gsem0, gsem1)
    ssem = (ssem0, ssem1)

    def fire_gather(cc, b):
        pltpu.async_copy(table_hbm.at[idx_v.at[cc]], rows[b], gsem[b])

    def wait_gather(cc, b):
        pltpu.make_async_copy(table_hbm.at[idx_v.at[cc]], rows[b],
                              gsem[b]).wait()

    def out_ref(cc):
        return out_hbm.at[pl.ds((base + cc) * L, L), pl.ds(0, D)]

    fire_gather(0, 0)

    def chunk_body(cc, b):
        @pl.when(cc >= 1)
        def _():
            pltpu.make_async_copy(rows[1 - b].at[:, pl.ds(0, D)],
                                  out_ref(cc - 1), ssem[1 - b]).wait()

        @pl.when(cc + 1 < NCHUNK)
        def _():
            fire_gather(cc + 1, 1 - b)

        wait_gather(cc, b)
        pltpu.async_copy(rows[b].at[:, pl.ds(0, D)], out_ref(cc), ssem[b])

    def outer(i, carry):
        cc = i * 2
        chunk_body(cc, 0)
        chunk_body(cc + 1, 1)
        return carry

    lax.fori_loop(0, NCHUNK // 2, outer, 0)

    pltpu.make_async_copy(rows[1].at[:, pl.ds(0, D)], out_ref(NCHUNK - 1),
                          ssem[1]).wait()


def kernel(x, table):
    tp = jnp.pad(table, ((0, 0), (0, DP - D)))
    outp = _embed_sc(x.astype(jnp.int32), tp)
    return outp.reshape(B, L, DP)[:, :, :D]
